# baseline (speedup 1.0000x reference)
import functools

import jax
import jax.numpy as jnp
from jax import lax
from jax.experimental import pallas as pl
from jax.experimental.pallas import tpu as pltpu
from jax.experimental.pallas import tpu_sc as plsc

_N = 200000
_D = 256
_B = 32
_NC = 2
_NS = 16
_NW = _NC * _NS
_C = 400
_NCHUNK = _N // _C
_FULL = _NCHUNK // _NW
_EXTRA = _NCHUNK % _NW


def _softmax_body(v_ref, o_ref):
    v = v_ref[...]
    v = v - jnp.max(v, axis=1, keepdims=True)
    e = jnp.exp(v)
    o_ref[...] = e / jnp.sum(e, axis=1, keepdims=True)


def _tc_softmax(vecter):
    return pl.pallas_call(
        _softmax_body,
        out_shape=jax.ShapeDtypeStruct((_B, _D), jnp.float32),
    )(vecter)


def _sc_body(feat_hbm, idx_hbm, vsm_hbm, out_hbm, tab_v, idx_v, feat_v, sem):
    wid = lax.axis_index("s") * _NC + lax.axis_index("c")
    pltpu.sync_copy(vsm_hbm, tab_v)
    nchunks = _FULL + jnp.where(wid < _EXTRA, 1, 0)

    def chunk_body(k, _):
        c = wid + _NW * k
        rbase = c * _C
        pltpu.sync_copy(idx_hbm.at[pl.ds(rbase, _C)], idx_v)
        pltpu.sync_copy(feat_hbm.at[pl.ds(rbase * _D, _C * _D)], feat_v)

        def group_body(g, _):
            idxg = idx_v[pl.ds(g * 16, 16)]
            for r in range(16):
                off = idxg[r] * _D
                pbase = (g * 16 + r) * _D
                for j in range(_D // 16):
                    a = feat_v[pl.ds(pbase + j * 16, 16)]
                    b = tab_v[pl.ds(off + j * 16, 16)]
                    feat_v[pl.ds(pbase + j * 16, 16)] = a * b
            return 0

        lax.fori_loop(0, _C // 16, group_body, 0)
        pltpu.sync_copy(feat_v, out_hbm.at[pl.ds(rbase * _D, _C * _D)])
        return 0

    lax.fori_loop(0, nchunks, chunk_body, 0)


def kernel(features, point_idx, vecter):
    vsm = _tc_softmax(vecter)
    feat_flat = features.reshape(_N * _D)
    vsm_flat = vsm.reshape(_B * _D)
    idx = point_idx.astype(jnp.int32)

    mesh = plsc.VectorSubcoreMesh(core_axis_name="c", subcore_axis_name="s")
    out_flat = pl.kernel(
        _sc_body,
        out_type=jax.ShapeDtypeStruct((_N * _D,), jnp.float32),
        mesh=mesh,
        scratch_types=[
            pltpu.VMEM((_B * _D,), jnp.float32),
            pltpu.VMEM((_C,), jnp.int32),
            pltpu.VMEM((_C * _D,), jnp.float32),
            pltpu.SemaphoreType.DMA,
        ],
    )(feat_flat, idx, vsm_flat)
    return out_flat.reshape(_N, _D)


# speedup vs baseline: 1.0259x; 1.0259x over previous
import functools

import jax
import jax.numpy as jnp
from jax import lax
from jax.experimental import pallas as pl
from jax.experimental.pallas import tpu as pltpu
from jax.experimental.pallas import tpu_sc as plsc

_N = 200000
_D = 256
_B = 32
_NC = 2
_NS = 16
_NW = _NC * _NS
_C = 160
_CD = _C * _D
_NCHUNK = _N // _C
_FULL = _NCHUNK // _NW
_EXTRA = _NCHUNK % _NW


def _softmax_body(v_ref, o_ref):
    v = v_ref[...]
    v = v - jnp.max(v, axis=1, keepdims=True)
    e = jnp.exp(v)
    o_ref[...] = e / jnp.sum(e, axis=1, keepdims=True)


def _tc_softmax(vecter):
    return pl.pallas_call(
        _softmax_body,
        out_shape=jax.ShapeDtypeStruct((_B, _D), jnp.float32),
    )(vecter)


def _sc_body(feat_hbm, idx_hbm, vsm_hbm, out_hbm,
             tab_v, idx_v, feat_v, isem, fsem, wsem):
    wid = lax.axis_index("s") * _NC + lax.axis_index("c")
    nchunks = _FULL + jnp.where(wid < _EXTRA, 1, 0)

    pltpu.sync_copy(vsm_hbm, tab_v)
    base0 = wid * _C
    pltpu.async_copy(idx_hbm.at[pl.ds(base0, _C)], idx_v.at[pl.ds(0, _C)], isem)
    pltpu.async_copy(feat_hbm.at[pl.ds(base0 * _D, _CD)],
                     feat_v.at[pl.ds(0, _CD)], fsem)

    def chunk_body(k, _):
        b = lax.rem(k, 2)
        nb = 1 - b
        have_next = k + 1 < nchunks

        @pl.when(jnp.logical_and(k >= 1, have_next))
        def _():
            pltpu.make_async_copy(feat_v.at[pl.ds(nb * _CD, _CD)],
                                  out_hbm.at[pl.ds(0, _CD)], wsem).wait()

        @pl.when(have_next)
        def _():
            nbase = (wid + _NW * (k + 1)) * _C
            pltpu.async_copy(idx_hbm.at[pl.ds(nbase, _C)],
                             idx_v.at[pl.ds(nb * _C, _C)], isem)
            pltpu.async_copy(feat_hbm.at[pl.ds(nbase * _D, _CD)],
                             feat_v.at[pl.ds(nb * _CD, _CD)], fsem)

        pltpu.make_async_copy(idx_hbm.at[pl.ds(0, _C)],
                              idx_v.at[pl.ds(b * _C, _C)], isem).wait()
        pltpu.make_async_copy(feat_hbm.at[pl.ds(0, _CD)],
                              feat_v.at[pl.ds(b * _CD, _CD)], fsem).wait()

        cbase = b * _CD

        @plsc.parallel_loop(0, _C // 16)
        def _group(g):
            idxg = idx_v[pl.ds(b * _C + g * 16, 16)]
            for r in range(16):
                off = idxg[r] * _D
                pbase = cbase + (g * 16 + r) * _D
                for j in range(_D // 16):
                    a = feat_v[pl.ds(pbase + j * 16, 16)]
                    t = tab_v[pl.ds(off + j * 16, 16)]
                    feat_v[pl.ds(pbase + j * 16, 16)] = a * t

        base = (wid + _NW * k) * _C
        pltpu.async_copy(feat_v.at[pl.ds(cbase, _CD)],
                         out_hbm.at[pl.ds(base * _D, _CD)], wsem)
        return 0

    lax.fori_loop(0, nchunks, chunk_body, 0)
    pltpu.make_async_copy(feat_v.at[pl.ds(0, _CD)],
                          out_hbm.at[pl.ds(0, _CD)], wsem).wait()
    pltpu.make_async_copy(feat_v.at[pl.ds(_CD, _CD)],
                          out_hbm.at[pl.ds(0, _CD)], wsem).wait()


def kernel(features, point_idx, vecter):
    vsm = _tc_softmax(vecter)
    feat_flat = features.reshape(_N * _D)
    vsm_flat = vsm.reshape(_B * _D)
    idx = point_idx.astype(jnp.int32)

    mesh = plsc.VectorSubcoreMesh(core_axis_name="c", subcore_axis_name="s")
    out_flat = pl.kernel(
        _sc_body,
        out_type=jax.ShapeDtypeStruct((_N * _D,), jnp.float32),
        mesh=mesh,
        scratch_types=[
            pltpu.VMEM((_B * _D,), jnp.float32),
            pltpu.VMEM((2 * _C,), jnp.int32),
            pltpu.VMEM((2 * _CD,), jnp.float32),
            pltpu.SemaphoreType.DMA,
            pltpu.SemaphoreType.DMA,
            pltpu.SemaphoreType.DMA,
        ],
    )(feat_flat, idx, vsm_flat)
    return out_flat.reshape(_N, _D)
